# speedup vs baseline: 1.4347x; 1.4347x over previous
import functools

import jax
import jax.numpy as jnp
from jax import lax
from jax.experimental import pallas as pl
from jax.experimental.pallas import tpu as pltpu
from jax.experimental.pallas import tpu_sc as plsc

VOCAB = 30522
HIDDEN = 768
BATCH = 4
SEQ = 2048
TOTAL = BATCH * SEQ

NUM_CORES = 2
NUM_SUBCORES = 16
NUM_WORKERS = NUM_CORES * NUM_SUBCORES
PER_WORKER = TOTAL // NUM_WORKERS

CHUNK = 64
NCHUNK = PER_WORKER // CHUNK
NBUF = 2

_mesh = plsc.VectorSubcoreMesh(core_axis_name="c", subcore_axis_name="s")


@functools.partial(
    pl.kernel,
    mesh=_mesh,
    out_type=jax.ShapeDtypeStruct((TOTAL, HIDDEN), jnp.float32),
    scratch_types=[
        pltpu.VMEM((PER_WORKER,), jnp.int32),
        pltpu.VMEM((NBUF, CHUNK, HIDDEN), jnp.float32),
        pltpu.SemaphoreType.DMA,
        pltpu.SemaphoreType.DMA,
        pltpu.SemaphoreType.DMA,
    ],
)
def _embed_lookup(idx_hbm, table_hbm, out_hbm, idx_v, rows_v, gsem, wsem0, wsem1):
    wid = lax.axis_index("s") * NUM_CORES + lax.axis_index("c")
    base = wid * PER_WORKER
    pltpu.sync_copy(idx_hbm.at[pl.ds(base, PER_WORKER)], idx_v)

    wsems = (wsem0, wsem1)

    def gather(c):
        return pltpu.async_copy(
            table_hbm.at[idx_v.at[pl.ds(c * CHUNK, CHUNK)]],
            rows_v.at[c % NBUF],
            gsem,
        )

    def write(c):
        return pltpu.async_copy(
            rows_v.at[c % NBUF],
            out_hbm.at[pl.ds(base + c * CHUNK, CHUNK)],
            wsems[c % NBUF],
        )

    gd = [None] * NCHUNK
    wd = [None] * NCHUNK
    gd[0] = gather(0)
    for c in range(NCHUNK):
        gd[c].wait()
        wd[c] = write(c)
        if c + 1 < NCHUNK:
            if c >= 1:
                wd[c - 1].wait()
            gd[c + 1] = gather(c + 1)
    wd[NCHUNK - 2].wait()
    wd[NCHUNK - 1].wait()


def kernel(input_ids, word_embeddings):
    idx = input_ids.reshape(-1).astype(jnp.int32)
    out = _embed_lookup(idx, word_embeddings)
    return out.reshape(input_ids.shape + (word_embeddings.shape[-1],))


# speedup vs baseline: 1.5041x; 1.0484x over previous
import functools

import jax
import jax.numpy as jnp
from jax import lax
from jax.experimental import pallas as pl
from jax.experimental.pallas import tpu as pltpu
from jax.experimental.pallas import tpu_sc as plsc

VOCAB = 30522
HIDDEN = 768
BATCH = 4
SEQ = 2048
TOTAL = BATCH * SEQ

NUM_CORES = 2
NUM_SUBCORES = 16
NUM_WORKERS = NUM_CORES * NUM_SUBCORES
PER_WORKER = TOTAL // NUM_WORKERS

CHUNK = 32
NCHUNK = PER_WORKER // CHUNK
NBUF = 4

_mesh = plsc.VectorSubcoreMesh(core_axis_name="c", subcore_axis_name="s")


@functools.partial(
    pl.kernel,
    mesh=_mesh,
    out_type=jax.ShapeDtypeStruct((TOTAL, HIDDEN), jnp.float32),
    scratch_types=[
        pltpu.VMEM((PER_WORKER,), jnp.int32),
        pltpu.VMEM((NBUF, CHUNK, HIDDEN), jnp.float32),
    ]
    + [pltpu.SemaphoreType.DMA] * (2 * NBUF),
)
def _embed_lookup(idx_hbm, table_hbm, out_hbm, idx_v, rows_v, *sems):
    gsems = sems[:NBUF]
    wsems = sems[NBUF:]
    wid = lax.axis_index("s") * NUM_CORES + lax.axis_index("c")
    base = wid * PER_WORKER
    pltpu.sync_copy(idx_hbm.at[pl.ds(base, PER_WORKER)], idx_v)

    def gather(c):
        return pltpu.async_copy(
            table_hbm.at[idx_v.at[pl.ds(c * CHUNK, CHUNK)]],
            rows_v.at[c % NBUF],
            gsems[c % NBUF],
        )

    def write(c):
        return pltpu.async_copy(
            rows_v.at[c % NBUF],
            out_hbm.at[pl.ds(base + c * CHUNK, CHUNK)],
            wsems[c % NBUF],
        )

    gd = [None] * NCHUNK
    wd = [None] * NCHUNK
    for c in range(min(NBUF - 1, NCHUNK)):
        gd[c] = gather(c)
    for c in range(NCHUNK):
        gd[c].wait()
        wd[c] = write(c)
        n = c + NBUF - 1
        if n < NCHUNK:
            if n - NBUF >= 0:
                wd[n - NBUF].wait()
            gd[n] = gather(n)
    for c in range(max(0, NCHUNK - NBUF), NCHUNK):
        wd[c].wait()


def kernel(input_ids, word_embeddings):
    idx = input_ids.reshape(-1).astype(jnp.int32)
    out = _embed_lookup(idx, word_embeddings)
    return out.reshape(input_ids.shape + (word_embeddings.shape[-1],))


# speedup vs baseline: 1.5102x; 1.0040x over previous
import functools

import jax
import jax.numpy as jnp
from jax import lax
from jax.experimental import pallas as pl
from jax.experimental.pallas import tpu as pltpu
from jax.experimental.pallas import tpu_sc as plsc

VOCAB = 30522
HIDDEN = 768
BATCH = 4
SEQ = 2048
TOTAL = BATCH * SEQ

NUM_CORES = 2
NUM_SUBCORES = 16
NUM_WORKERS = NUM_CORES * NUM_SUBCORES
PER_WORKER = TOTAL // NUM_WORKERS

CHUNK = 32
NCHUNK = PER_WORKER // CHUNK
NBUF = 4

_mesh = plsc.VectorSubcoreMesh(core_axis_name="c", subcore_axis_name="s")


WORKERS_PER_BATCH = NUM_WORKERS // BATCH
SEQ_PER_WORKER = SEQ // WORKERS_PER_BATCH


@functools.partial(
    pl.kernel,
    mesh=_mesh,
    out_type=jax.ShapeDtypeStruct((BATCH, SEQ, HIDDEN), jnp.float32),
    scratch_types=[
        pltpu.VMEM((PER_WORKER,), jnp.int32),
        pltpu.VMEM((NBUF, CHUNK, HIDDEN), jnp.float32),
    ]
    + [pltpu.SemaphoreType.DMA] * (2 * NBUF),
)
def _embed_lookup(idx_hbm, table_hbm, out_hbm, idx_v, rows_v, *sems):
    gsems = sems[:NBUF]
    wsems = sems[NBUF:]
    wid = lax.axis_index("s") * NUM_CORES + lax.axis_index("c")
    b = wid // WORKERS_PER_BATCH
    s0 = (wid % WORKERS_PER_BATCH) * SEQ_PER_WORKER
    pltpu.sync_copy(idx_hbm.at[b, pl.ds(s0, SEQ_PER_WORKER)], idx_v)

    def gather(c):
        return pltpu.async_copy(
            table_hbm.at[idx_v.at[pl.ds(c * CHUNK, CHUNK)]],
            rows_v.at[c % NBUF],
            gsems[c % NBUF],
        )

    def write(c):
        return pltpu.async_copy(
            rows_v.at[c % NBUF],
            out_hbm.at[b, pl.ds(s0 + c * CHUNK, CHUNK)],
            wsems[c % NBUF],
        )

    gd = [None] * NCHUNK
    wd = [None] * NCHUNK
    for c in range(min(NBUF - 1, NCHUNK)):
        gd[c] = gather(c)
    for c in range(NCHUNK):
        gd[c].wait()
        wd[c] = write(c)
        n = c + NBUF - 1
        if n < NCHUNK:
            if n - NBUF >= 0:
                wd[n - NBUF].wait()
            gd[n] = gather(n)
    for c in range(max(0, NCHUNK - NBUF), NCHUNK):
        wd[c].wait()


def kernel(input_ids, word_embeddings):
    return _embed_lookup(input_ids, word_embeddings)


# speedup vs baseline: 1.5214x; 1.0074x over previous
import functools

import jax
import jax.numpy as jnp
from jax import lax
from jax.experimental import pallas as pl
from jax.experimental.pallas import tpu as pltpu
from jax.experimental.pallas import tpu_sc as plsc

VOCAB = 30522
HIDDEN = 768
BATCH = 4
SEQ = 2048
TOTAL = BATCH * SEQ

NUM_CORES = 2
NUM_SUBCORES = 16
NUM_WORKERS = NUM_CORES * NUM_SUBCORES
PER_WORKER = TOTAL // NUM_WORKERS

CHUNK = 32
NCHUNK = PER_WORKER // CHUNK
NBUF = 5

_mesh = plsc.VectorSubcoreMesh(core_axis_name="c", subcore_axis_name="s")


WORKERS_PER_BATCH = NUM_WORKERS // BATCH
SEQ_PER_WORKER = SEQ // WORKERS_PER_BATCH


@functools.partial(
    pl.kernel,
    mesh=_mesh,
    out_type=jax.ShapeDtypeStruct((BATCH, SEQ, HIDDEN), jnp.float32),
    scratch_types=[
        pltpu.VMEM((PER_WORKER,), jnp.int32),
        pltpu.VMEM((NBUF, CHUNK, HIDDEN), jnp.float32),
    ]
    + [pltpu.SemaphoreType.DMA] * (2 * NBUF),
)
def _embed_lookup(idx_hbm, table_hbm, out_hbm, idx_v, rows_v, *sems):
    gsems = sems[:NBUF]
    wsems = sems[NBUF:]
    wid = lax.axis_index("s") * NUM_CORES + lax.axis_index("c")
    b = wid // WORKERS_PER_BATCH
    s0 = (wid % WORKERS_PER_BATCH) * SEQ_PER_WORKER
    pltpu.sync_copy(idx_hbm.at[b, pl.ds(s0, SEQ_PER_WORKER)], idx_v)

    def gather(c):
        return pltpu.async_copy(
            table_hbm.at[idx_v.at[pl.ds(c * CHUNK, CHUNK)]],
            rows_v.at[c % NBUF],
            gsems[c % NBUF],
        )

    def write(c):
        return pltpu.async_copy(
            rows_v.at[c % NBUF],
            out_hbm.at[b, pl.ds(s0 + c * CHUNK, CHUNK)],
            wsems[c % NBUF],
        )

    gd = [None] * NCHUNK
    wd = [None] * NCHUNK
    for c in range(min(NBUF - 1, NCHUNK)):
        gd[c] = gather(c)
    for c in range(NCHUNK):
        gd[c].wait()
        wd[c] = write(c)
        n = c + NBUF - 1
        if n < NCHUNK:
            if n - NBUF >= 0:
                wd[n - NBUF].wait()
            gd[n] = gather(n)
    for c in range(max(0, NCHUNK - NBUF), NCHUNK):
        wd[c].wait()


def kernel(input_ids, word_embeddings):
    return _embed_lookup(input_ids, word_embeddings)
